# baseline (speedup 1.0000x reference)
import jax
import jax.numpy as jnp
from jax.experimental import pallas as pl
from jax.experimental.pallas import tpu as pltpu

_TM = 512
_N_CORES = 2


def _pick_tm(n_pad):
    for tm in (_TM, 256, 128):
        if n_pad % (tm * _N_CORES) == 0:
            return tm
    return n_pad


def _vmem_limit():
    return 56 * 1024 * 1024


def _l0_kernel(a_ref, xs_ref, xr_ref, wl0_ref, wr0_ref, b0_ref,
               wl1_ref, wr1_ref, b1_ref, y_ref, p_ref, agg_ref):
    bf16 = jnp.bfloat16
    n_types, tm, k = a_ref.shape
    din, dh = wl0_ref.shape[1], wl0_ref.shape[2]
    j = pl.program_id(1)
    n_j = pl.num_programs(1) - 1

    @pl.when(j < n_j)
    def _():
        a = a_ref[...].reshape(n_types * tm, k)
        agg_ref[j % 2] = jnp.dot(a, xs_ref[...],
                                 preferred_element_type=jnp.float32)

    if True:
        zpad = jnp.zeros((din, dh), bf16)
        w0_big = jnp.concatenate([
            jnp.concatenate([wl0_ref[0].astype(bf16), zpad], axis=1),
            jnp.concatenate([zpad, wl0_ref[1].astype(bf16)], axis=1),
            jnp.concatenate([wr0_ref[0].astype(bf16), wr0_ref[1].astype(bf16)],
                            axis=1),
        ], axis=0)
        b0_cat = jnp.concatenate([b0_ref[0], b0_ref[1]], axis=1)
        w1_big = jnp.concatenate(
            [wl1_ref[0], wl1_ref[1], wr1_ref[0], wr1_ref[1]],
            axis=1).astype(bf16)
        b1_cat = jnp.concatenate([b1_ref[0], b1_ref[1]], axis=1)

        agg = agg_ref[(j - 1) % 2]
        parts = [agg[t * tm:(t + 1) * tm].astype(bf16) for t in range(n_types)]
        parts.append(xr_ref[...].astype(bf16))
        z = jnp.concatenate(parts, axis=1)
        h = jnp.dot(z, w0_big, preferred_element_type=jnp.float32) + b0_cat
        emb0 = jnp.maximum(h, 0.0).astype(bf16)
        yp = jnp.dot(emb0, w1_big, preferred_element_type=jnp.float32)
        width = y_ref.shape[1]
        y_ref[...] = yp[:, :width].astype(bf16)
        p_ref[...] = (yp[:, width:] + b1_cat).astype(bf16)


def _l1_kernel(a_ref, y_ref, p_ref, w1_ref, b1_ref, w2_ref, b2_ref,
               emb_ref, pred_ref, acc_ref):
    bf16 = jnp.bfloat16
    n_types, tm, k = a_ref.shape
    dh = y_ref.shape[1] // n_types
    j = pl.program_id(1)
    n_j = pl.num_programs(1) - 1

    @pl.when(j < n_j)
    def _():
        for t in range(n_types):
            acc_ref[j % 2, :, t * dh:(t + 1) * dh] = jnp.dot(
                a_ref[t], y_ref[:, t * dh:(t + 1) * dh],
                preferred_element_type=jnp.float32)

    if True:
        h = jnp.maximum(acc_ref[(j - 1) % 2]
                        + p_ref[...].astype(jnp.float32), 0.0)
        emb_ref[...] = h
        hh = jnp.maximum(
            jnp.dot(h.astype(bf16), w1_ref[...].astype(bf16),
                    preferred_element_type=jnp.float32) + b1_ref[...], 0.0)
        pred_ref[...] = (jnp.dot(hh.astype(bf16), w2_ref[...].astype(bf16),
                                 preferred_element_type=jnp.float32)
                         + b2_ref[...])


def _compiler_params():
    return pltpu.CompilerParams(
        dimension_semantics=("parallel", "arbitrary"),
        vmem_limit_bytes=_vmem_limit())


def kernel(x, w_l_0, w_r_0, b_0, w_l_1, w_r_1, b_1, w1, b1, w2, b2, A_hat):
    n_types, n_pad, _ = A_hat.shape
    num_nodes, din = x.shape
    dh = w_l_0.shape[2]
    width = n_types * dh
    d_out = w2.shape[1]
    d_hid = w1.shape[1]
    tm = _pick_tm(n_pad)
    n_tiles = n_pad // tm
    n_j = n_tiles // _N_CORES
    grid = (_N_CORES, n_j + 1)

    xb = x.astype(jnp.bfloat16)
    if n_pad != num_nodes:
        xb = jnp.zeros((n_pad, din), jnp.bfloat16).at[:num_nodes].set(xb)

    def a_idx(c, j):
        return (0, c * n_j + jnp.minimum(j, n_j - 1), 0)

    def epi_idx(c, j):
        return (c * n_j + jnp.maximum(j, 1) - 1, 0)

    full = lambda shape: pl.BlockSpec(
        shape, lambda c, j: tuple(0 for _ in shape))

    y1, p1 = pl.pallas_call(
        _l0_kernel,
        out_shape=(jax.ShapeDtypeStruct((n_pad, width), jnp.bfloat16),
                   jax.ShapeDtypeStruct((n_pad, width), jnp.bfloat16)),
        grid=grid,
        in_specs=[
            pl.BlockSpec((n_types, tm, n_pad), a_idx),
            full((n_pad, din)),
            pl.BlockSpec((tm, din), epi_idx),
            full((n_types, din, dh)),
            full((n_types, din, dh)),
            full((n_types, 1, dh)),
            full((n_types, width, dh)),
            full((n_types, width, dh)),
            full((n_types, 1, dh)),
        ],
        out_specs=(pl.BlockSpec((tm, width), epi_idx),
                   pl.BlockSpec((tm, width), epi_idx)),
        scratch_shapes=[pltpu.VMEM((2, n_types * tm, din), jnp.float32)],
        compiler_params=_compiler_params(),
    )(A_hat, xb, xb, w_l_0, w_r_0, b_0, w_l_1, w_r_1, b_1)

    emb, pred = pl.pallas_call(
        _l1_kernel,
        out_shape=(jax.ShapeDtypeStruct((n_pad, width), jnp.float32),
                   jax.ShapeDtypeStruct((n_pad, d_out), jnp.float32)),
        grid=grid,
        in_specs=[
            pl.BlockSpec((n_types, tm, n_pad), a_idx),
            full((n_pad, width)),
            pl.BlockSpec((tm, width), epi_idx),
            full((width, d_hid)),
            full((1, d_hid)),
            full((d_hid, d_out)),
            full((1, d_out)),
        ],
        out_specs=(pl.BlockSpec((tm, width), epi_idx),
                   pl.BlockSpec((tm, d_out), epi_idx)),
        scratch_shapes=[pltpu.VMEM((2, tm, width), jnp.float32)],
        compiler_params=_compiler_params(),
    )(A_hat, y1, p1, w1, b1, w2, b2)

    if num_nodes != n_pad:
        emb = emb[:num_nodes]
        pred = pred[:num_nodes]
    return emb, pred


# speedup vs baseline: 1.0249x; 1.0249x over previous
import jax
import jax.numpy as jnp
from jax.experimental import pallas as pl
from jax.experimental.pallas import tpu as pltpu

_TM = 512
_N_CORES = 2


def _pick_tm(n_pad):
    for tm in (_TM, 256, 128):
        if n_pad % (tm * _N_CORES) == 0:
            return tm
    return n_pad


def _vmem_limit():
    return 56 * 1024 * 1024


def _l0_kernel(a_ref, xs_ref, xr_ref, wl0_ref, wr0_ref, b0_ref,
               wl1_ref, wr1_ref, b1_ref, y_ref, p_ref, agg0_ref, agg1_ref):
    bf16 = jnp.bfloat16
    n_types, tm, k = a_ref.shape
    din, dh = wl0_ref.shape[1], wl0_ref.shape[2]
    j = pl.program_id(1)
    n_j = pl.num_programs(1) - 1

    for par, dst in ((0, agg0_ref), (1, agg1_ref)):
        @pl.when(jnp.logical_and(j < n_j, j % 2 == par))
        def _(dst=dst):
            a = a_ref[...].reshape(n_types * tm, k)
            dst[...] = jnp.dot(a, xs_ref[...],
                               preferred_element_type=jnp.float32)

    def _epilogue(agg_ref):
        zpad = jnp.zeros((din, dh), bf16)
        w0_big = jnp.concatenate([
            jnp.concatenate([wl0_ref[0].astype(bf16), zpad], axis=1),
            jnp.concatenate([zpad, wl0_ref[1].astype(bf16)], axis=1),
            jnp.concatenate([wr0_ref[0].astype(bf16), wr0_ref[1].astype(bf16)],
                            axis=1),
        ], axis=0)
        b0_cat = jnp.concatenate([b0_ref[0], b0_ref[1]], axis=1)
        w1_big = jnp.concatenate(
            [wl1_ref[0], wl1_ref[1], wr1_ref[0], wr1_ref[1]],
            axis=1).astype(bf16)
        b1_cat = jnp.concatenate([b1_ref[0], b1_ref[1]], axis=1)

        agg = agg_ref[...]
        parts = [agg[t * tm:(t + 1) * tm].astype(bf16) for t in range(n_types)]
        parts.append(xr_ref[...].astype(bf16))
        z = jnp.concatenate(parts, axis=1)
        h = jnp.dot(z, w0_big, preferred_element_type=jnp.float32) + b0_cat
        emb0 = jnp.maximum(h, 0.0).astype(bf16)
        yp = jnp.dot(emb0, w1_big, preferred_element_type=jnp.float32)
        width = y_ref.shape[1]
        y_ref[...] = yp[:, :width].astype(bf16)
        p_ref[...] = (yp[:, width:] + b1_cat).astype(bf16)

    for par, srcb in ((1, agg0_ref), (0, agg1_ref)):
        @pl.when(jnp.logical_and(j > 0, j % 2 == par))
        def _(srcb=srcb):
            _epilogue(srcb)


def _l1_kernel(a_ref, y_ref, p_ref, w1_ref, b1_ref, w2_ref, b2_ref,
               emb_ref, pred_ref, acc0_ref, acc1_ref):
    bf16 = jnp.bfloat16
    n_types, tm, k = a_ref.shape
    dh = y_ref.shape[1] // n_types
    j = pl.program_id(1)
    n_j = pl.num_programs(1) - 1

    for par, dst in ((0, acc0_ref), (1, acc1_ref)):
        @pl.when(jnp.logical_and(j < n_j, j % 2 == par))
        def _(dst=dst):
            for t in range(n_types):
                dst[:, t * dh:(t + 1) * dh] = jnp.dot(
                    a_ref[t], y_ref[:, t * dh:(t + 1) * dh],
                    preferred_element_type=jnp.float32)

    def _epilogue(acc_ref):
        h = jnp.maximum(acc_ref[...]
                        + p_ref[...].astype(jnp.float32), 0.0)
        emb_ref[...] = h
        hh = jnp.maximum(
            jnp.dot(h.astype(bf16), w1_ref[...].astype(bf16),
                    preferred_element_type=jnp.float32) + b1_ref[...], 0.0)
        pred_ref[...] = (jnp.dot(hh.astype(bf16), w2_ref[...].astype(bf16),
                                 preferred_element_type=jnp.float32)
                         + b2_ref[...])

    for par, srcb in ((1, acc0_ref), (0, acc1_ref)):
        @pl.when(jnp.logical_and(j > 0, j % 2 == par))
        def _(srcb=srcb):
            _epilogue(srcb)


def _compiler_params():
    return pltpu.CompilerParams(
        dimension_semantics=("parallel", "arbitrary"),
        vmem_limit_bytes=_vmem_limit())


def kernel(x, w_l_0, w_r_0, b_0, w_l_1, w_r_1, b_1, w1, b1, w2, b2, A_hat):
    n_types, n_pad, _ = A_hat.shape
    num_nodes, din = x.shape
    dh = w_l_0.shape[2]
    width = n_types * dh
    d_out = w2.shape[1]
    d_hid = w1.shape[1]
    tm = _pick_tm(n_pad)
    n_tiles = n_pad // tm
    n_j = n_tiles // _N_CORES
    grid = (_N_CORES, n_j + 1)

    xb = x.astype(jnp.bfloat16)
    if n_pad != num_nodes:
        xb = jnp.zeros((n_pad, din), jnp.bfloat16).at[:num_nodes].set(xb)

    def a_idx(c, j):
        return (0, c * n_j + jnp.minimum(j, n_j - 1), 0)

    def epi_idx(c, j):
        return (c * n_j + jnp.maximum(j, 1) - 1, 0)

    full = lambda shape: pl.BlockSpec(
        shape, lambda c, j: tuple(0 for _ in shape))

    y1, p1 = pl.pallas_call(
        _l0_kernel,
        out_shape=(jax.ShapeDtypeStruct((n_pad, width), jnp.bfloat16),
                   jax.ShapeDtypeStruct((n_pad, width), jnp.bfloat16)),
        grid=grid,
        in_specs=[
            pl.BlockSpec((n_types, tm, n_pad), a_idx),
            full((n_pad, din)),
            pl.BlockSpec((tm, din), epi_idx),
            full((n_types, din, dh)),
            full((n_types, din, dh)),
            full((n_types, 1, dh)),
            full((n_types, width, dh)),
            full((n_types, width, dh)),
            full((n_types, 1, dh)),
        ],
        out_specs=(pl.BlockSpec((tm, width), epi_idx),
                   pl.BlockSpec((tm, width), epi_idx)),
        scratch_shapes=[pltpu.VMEM((n_types * tm, din), jnp.float32),
                        pltpu.VMEM((n_types * tm, din), jnp.float32)],
        compiler_params=_compiler_params(),
    )(A_hat, xb, xb, w_l_0, w_r_0, b_0, w_l_1, w_r_1, b_1)

    emb, pred = pl.pallas_call(
        _l1_kernel,
        out_shape=(jax.ShapeDtypeStruct((n_pad, width), jnp.float32),
                   jax.ShapeDtypeStruct((n_pad, d_out), jnp.float32)),
        grid=grid,
        in_specs=[
            pl.BlockSpec((n_types, tm, n_pad), a_idx),
            full((n_pad, width)),
            pl.BlockSpec((tm, width), epi_idx),
            full((width, d_hid)),
            full((1, d_hid)),
            full((d_hid, d_out)),
            full((1, d_out)),
        ],
        out_specs=(pl.BlockSpec((tm, width), epi_idx),
                   pl.BlockSpec((tm, d_out), epi_idx)),
        scratch_shapes=[pltpu.VMEM((tm, width), jnp.float32),
                        pltpu.VMEM((tm, width), jnp.float32)],
        compiler_params=_compiler_params(),
    )(A_hat, y1, p1, w1, b1, w2, b2)

    if num_nodes != n_pad:
        emb = emb[:num_nodes]
        pred = pred[:num_nodes]
    return emb, pred
